# baseline (speedup 1.0000x reference)
import jax
import jax.numpy as jnp
from jax import lax
from jax.experimental import pallas as pl
from jax.experimental.pallas import tpu as pltpu
from jax.experimental.pallas import tpu_sc as plsc

B, S, D = 4, 4096, 1024
MAX_ROWS = 64
N = B * S
NC, NS = 2, 16
NW = NC * NS
TPW = N // NW
C = 4
NBUF = 8
NCHUNK = TPW // C
NGROUP = NCHUNK // NBUF
LANES = 16
GROUPS = D // LANES


def _body(tokens_hbm, idx_hbm, table_hbm, out_hbm,
          idx_v, table_v, tin0, tin1, tin2, tin3, tin4, tin5, tin6, tin7,
          sem_t, in_sem0, in_sem1, in_sem2, in_sem3,
          in_sem4, in_sem5, in_sem6, in_sem7,
          out_sem0, out_sem1, out_sem2, out_sem3,
          out_sem4, out_sem5, out_sem6, out_sem7):
    wid = lax.axis_index("s") * NC + lax.axis_index("c")
    base = wid * TPW

    tins = (tin0, tin1, tin2, tin3, tin4, tin5, tin6, tin7)
    in_sems = (in_sem0, in_sem1, in_sem2, in_sem3,
               in_sem4, in_sem5, in_sem6, in_sem7)
    out_sems = (out_sem0, out_sem1, out_sem2, out_sem3,
                out_sem4, out_sem5, out_sem6, out_sem7)

    tbl_copy = pltpu.async_copy(table_hbm, table_v, sem_t)
    pltpu.sync_copy(idx_hbm.at[pl.ds(base, TPW)], idx_v.at[pl.ds(0, TPW)])

    def start_in(c, b):
        pltpu.async_copy(tokens_hbm.at[pl.ds(base + c * C, C)], tins[b],
                         in_sems[b])

    def start_out(c, b):
        pltpu.async_copy(tins[b], out_hbm.at[pl.ds(base + c * C, C)],
                         out_sems[b])

    def wait_in(b):
        pltpu.make_async_copy(tokens_hbm.at[pl.ds(base, C)], tins[b],
                              in_sems[b]).wait()

    def wait_out(b):
        pltpu.make_async_copy(tins[b], out_hbm.at[pl.ds(base, C)],
                              out_sems[b]).wait()

    def compute(c, b):
        tin = tins[b]

        def row_body(i, carry):
            row = idx_v[pl.ds(c * C + i, LANES)][0]
            for j0 in range(0, GROUPS, 32):
                gs = [table_v[row, pl.ds((j0 + j) * LANES, LANES)]
                      for j in range(32)]
                for j in range(32):
                    plsc.addupdate(tin.at[i, pl.ds((j0 + j) * LANES, LANES)],
                                   gs[j])
            return carry

        lax.fori_loop(0, C, row_body, 0)

    for b in range(NBUF // 2):
        start_in(b, b)
    tbl_copy.wait()

    HALF = NBUF // 2

    def turn(c, b, first, last):
        wait_in(b)
        bp = (b + HALF) % NBUF
        if not (last and b >= HALF):
            if not (first and b < HALF):
                wait_out(bp)
            start_in(c + HALF, bp)
        compute(c, b)
        start_out(c, b)

    def group(g, first, last):
        for b in range(NBUF):
            turn(g * NBUF + b, b, first, last)

    def mid(g, carry):
        group(g, False, False)
        return carry

    group(0, True, False)
    lax.fori_loop(1, NGROUP - 1, mid, 0)
    group(NGROUP - 1, False, True)

    for b in range(NBUF):
        wait_out(b)


@jax.jit
def _run(tokens2d, idx1d, table):
    mesh = plsc.VectorSubcoreMesh(core_axis_name="c", subcore_axis_name="s")
    f = pl.kernel(
        _body,
        out_type=jax.ShapeDtypeStruct((N, D), jnp.float32),
        mesh=mesh,
        compiler_params=pltpu.CompilerParams(needs_layout_passes=False),
        scratch_types=[
            pltpu.VMEM((TPW + LANES,), jnp.int32),
            pltpu.VMEM((MAX_ROWS, D), jnp.float32),
            pltpu.VMEM((C, D), jnp.float32),
            pltpu.VMEM((C, D), jnp.float32),
            pltpu.VMEM((C, D), jnp.float32),
            pltpu.VMEM((C, D), jnp.float32),
            pltpu.VMEM((C, D), jnp.float32),
            pltpu.VMEM((C, D), jnp.float32),
            pltpu.VMEM((C, D), jnp.float32),
            pltpu.VMEM((C, D), jnp.float32),
        ] + [pltpu.SemaphoreType.DMA] * 17,
    )
    return f(tokens2d, idx1d, table)


def kernel(tokens, example_positions, pos_table):
    tokens2d = tokens.reshape(N, D)
    idx1d = example_positions.reshape(N).astype(jnp.int32)
    out = _run(tokens2d, idx1d, pos_table)
    return out.reshape(B, S, D)


# speedup vs baseline: 1.0025x; 1.0025x over previous
import jax
import jax.numpy as jnp
from jax import lax
from jax.experimental import pallas as pl
from jax.experimental.pallas import tpu as pltpu
from jax.experimental.pallas import tpu_sc as plsc

B, S, D = 4, 4096, 1024
MAX_ROWS = 64
N = B * S
NC, NS = 2, 16
NW = NC * NS
TPW = N // NW
C = 8
NBUF = 4
NCHUNK = TPW // C
NGROUP = NCHUNK // NBUF
LANES = 16
GROUPS = D // LANES


def _body(tokens_hbm, idx_hbm, table_hbm, out_hbm,
          idx_v, table_v, tin0, tin1, tin2, tin3,
          sem_t, in_sem0, in_sem1, in_sem2, in_sem3,
          out_sem0, out_sem1, out_sem2, out_sem3):
    wid = lax.axis_index("s") * NC + lax.axis_index("c")
    base = wid * TPW

    tins = (tin0, tin1, tin2, tin3)
    in_sems = (in_sem0, in_sem1, in_sem2, in_sem3)
    out_sems = (out_sem0, out_sem1, out_sem2, out_sem3)

    tbl_copy = pltpu.async_copy(table_hbm, table_v, sem_t)
    pltpu.sync_copy(idx_hbm.at[pl.ds(base, TPW)], idx_v.at[pl.ds(0, TPW)])

    def start_in(c, b):
        pltpu.async_copy(tokens_hbm.at[pl.ds(base + c * C, C)], tins[b],
                         in_sems[b])

    def start_out(c, b):
        pltpu.async_copy(tins[b], out_hbm.at[pl.ds(base + c * C, C)],
                         out_sems[b])

    def wait_in(b):
        pltpu.make_async_copy(tokens_hbm.at[pl.ds(base, C)], tins[b],
                              in_sems[b]).wait()

    def wait_out(b):
        pltpu.make_async_copy(tins[b], out_hbm.at[pl.ds(base, C)],
                              out_sems[b]).wait()

    def compute(c, b):
        tin = tins[b]

        def row_body(i2, carry):
            i = i2 * 2
            rowa = idx_v[pl.ds(c * C + i, LANES)][0]
            rowb = idx_v[pl.ds(c * C + i + 1, LANES)][0]
            for j0 in range(0, GROUPS, 16):
                ga = [table_v[rowa, pl.ds((j0 + j) * LANES, LANES)]
                      for j in range(16)]
                gb = [table_v[rowb, pl.ds((j0 + j) * LANES, LANES)]
                      for j in range(16)]
                for j in range(16):
                    plsc.addupdate(tin.at[i, pl.ds((j0 + j) * LANES, LANES)],
                                   ga[j])
                for j in range(16):
                    plsc.addupdate(
                        tin.at[i + 1, pl.ds((j0 + j) * LANES, LANES)], gb[j])
            return carry

        lax.fori_loop(0, C // 2, row_body, 0)

    start_in(0, 0)
    start_in(1, 1)
    tbl_copy.wait()

    def turn(c, b, first, last):
        wait_in(b)
        bp = (b + 2) % NBUF
        if not (last and b >= 2):
            if not (first and b < 2):
                wait_out(bp)
            start_in(c + 2, bp)
        compute(c, b)
        start_out(c, b)

    def group(g, first, last):
        for b in range(NBUF):
            turn(g * NBUF + b, b, first, last)

    def mid(g, carry):
        group(g, False, False)
        return carry

    group(0, True, False)
    lax.fori_loop(1, NGROUP - 1, mid, 0)
    group(NGROUP - 1, False, True)

    for b in range(NBUF):
        wait_out(b)


@jax.jit
def _run(tokens2d, idx1d, table):
    mesh = plsc.VectorSubcoreMesh(core_axis_name="c", subcore_axis_name="s")
    f = pl.kernel(
        _body,
        out_type=jax.ShapeDtypeStruct((N, D), jnp.float32),
        mesh=mesh,
        compiler_params=pltpu.CompilerParams(needs_layout_passes=False),
        scratch_types=[
            pltpu.VMEM((TPW + LANES,), jnp.int32),
            pltpu.VMEM((MAX_ROWS, D), jnp.float32),
            pltpu.VMEM((C, D), jnp.float32),
            pltpu.VMEM((C, D), jnp.float32),
            pltpu.VMEM((C, D), jnp.float32),
            pltpu.VMEM((C, D), jnp.float32),
            pltpu.SemaphoreType.DMA,
            pltpu.SemaphoreType.DMA,
            pltpu.SemaphoreType.DMA,
            pltpu.SemaphoreType.DMA,
            pltpu.SemaphoreType.DMA,
            pltpu.SemaphoreType.DMA,
            pltpu.SemaphoreType.DMA,
            pltpu.SemaphoreType.DMA,
            pltpu.SemaphoreType.DMA,
        ],
    )
    return f(tokens2d, idx1d, table)


def kernel(tokens, example_positions, pos_table):
    tokens2d = tokens.reshape(N, D)
    idx1d = example_positions.reshape(N).astype(jnp.int32)
    out = _run(tokens2d, idx1d, pos_table)
    return out.reshape(B, S, D)


# speedup vs baseline: 1.0082x; 1.0057x over previous
import jax
import jax.numpy as jnp
from jax import lax
from jax.experimental import pallas as pl
from jax.experimental.pallas import tpu as pltpu
from jax.experimental.pallas import tpu_sc as plsc

B, S, D = 4, 4096, 1024
MAX_ROWS = 64
N = B * S
NC, NS = 2, 16
NW = NC * NS
TPW = N // NW
C = 16
NBUF = 3
NCHUNK = TPW // C
LANES = 16
GROUPS = D // LANES


def _body(tokens_hbm, idx_hbm, table_hbm, out_hbm,
          idx_v, table_v, tin0, tin1, tin2,
          sem_t, in_sem0, in_sem1, in_sem2, out_sem0, out_sem1, out_sem2):
    wid = lax.axis_index("s") * NC + lax.axis_index("c")
    base = wid * TPW

    tins = (tin0, tin1, tin2)
    in_sems = (in_sem0, in_sem1, in_sem2)
    out_sems = (out_sem0, out_sem1, out_sem2)

    tbl_copy = pltpu.async_copy(table_hbm, table_v, sem_t)
    pltpu.sync_copy(idx_hbm.at[pl.ds(base, TPW)], idx_v.at[pl.ds(0, TPW)])

    def start_in(c, b):
        pltpu.async_copy(tokens_hbm.at[pl.ds(base + c * C, C)], tins[b],
                         in_sems[b])

    def start_out(c, b):
        pltpu.async_copy(tins[b], out_hbm.at[pl.ds(base + c * C, C)],
                         out_sems[b])

    def wait_in(b):
        pltpu.make_async_copy(tokens_hbm.at[pl.ds(base, C)], tins[b],
                              in_sems[b]).wait()

    def wait_out(b):
        pltpu.make_async_copy(tins[b], out_hbm.at[pl.ds(base, C)],
                              out_sems[b]).wait()

    def compute(c, b):
        tin = tins[b]

        def row_body(i, carry):
            row = idx_v[pl.ds(c * C + i, LANES)][0]
            for j0 in range(0, GROUPS, 32):
                gs = [table_v[row, pl.ds((j0 + j) * LANES, LANES)]
                      for j in range(32)]
                for j in range(32):
                    plsc.addupdate(tin.at[i, pl.ds((j0 + j) * LANES, LANES)],
                                   gs[j])
            return carry

        lax.fori_loop(0, C, row_body, 0)

    start_in(0, 0)
    start_in(1, 1)
    tbl_copy.wait()

    def turn(c, b, wait_prev_out, refill):
        wait_in(b)
        compute(c, b)
        start_out(c, b)
        if refill:
            bp = (b + 2) % NBUF
            if wait_prev_out:
                wait_out(bp)
            start_in(c + 2, bp)

    turn(0, 0, False, True)
    turn(1, 1, True, True)

    def mid(k, carry):
        c = 3 * k + 2
        turn(c, 2, True, True)
        turn(c + 1, 0, True, True)
        turn(c + 2, 1, True, True)
        return carry

    lax.fori_loop(0, 9, mid, 0)

    turn(29, 2, True, True)
    turn(30, 0, True, False)
    turn(31, 1, True, False)

    for b in range(NBUF):
        wait_out(b)


@jax.jit
def _run(tokens2d, idx1d, table):
    mesh = plsc.VectorSubcoreMesh(core_axis_name="c", subcore_axis_name="s")
    f = pl.kernel(
        _body,
        out_type=jax.ShapeDtypeStruct((N, D), jnp.float32),
        mesh=mesh,
        compiler_params=pltpu.CompilerParams(needs_layout_passes=False),
        scratch_types=[
            pltpu.VMEM((TPW + LANES,), jnp.int32),
            pltpu.VMEM((MAX_ROWS, D), jnp.float32),
            pltpu.VMEM((C, D), jnp.float32),
            pltpu.VMEM((C, D), jnp.float32),
            pltpu.VMEM((C, D), jnp.float32),
        ] + [pltpu.SemaphoreType.DMA] * 7,
    )
    return f(tokens2d, idx1d, table)


def kernel(tokens, example_positions, pos_table):
    tokens2d = tokens.reshape(N, D)
    idx1d = example_positions.reshape(N).astype(jnp.int32)
    out = _run(tokens2d, idx1d, pos_table)
    return out.reshape(B, S, D)


# speedup vs baseline: 1.0384x; 1.0300x over previous
import jax
import jax.numpy as jnp
from jax import lax
from jax.experimental import pallas as pl
from jax.experimental.pallas import tpu as pltpu
from jax.experimental.pallas import tpu_sc as plsc

B, S, D = 4, 4096, 1024
MAX_ROWS = 64
N = B * S
NC, NS = 2, 16
NW = NC * NS
TPW = N // NW
C = 8
NBUF = 4
NCHUNK = TPW // C
NGROUP = NCHUNK // NBUF
LANES = 16
GROUPS = D // LANES


def _body(tokens_hbm, idx_hbm, table_hbm, out_hbm,
          idx_v, table_v, tin0, tin1, tin2, tin3,
          sem_t, in_sem0, in_sem1, in_sem2, in_sem3,
          out_sem0, out_sem1, out_sem2, out_sem3):
    wid = lax.axis_index("s") * NC + lax.axis_index("c")
    base = wid * TPW

    tins = (tin0, tin1, tin2, tin3)
    in_sems = (in_sem0, in_sem1, in_sem2, in_sem3)
    out_sems = (out_sem0, out_sem1, out_sem2, out_sem3)

    tbl_copy = pltpu.async_copy(table_hbm, table_v, sem_t)
    pltpu.sync_copy(idx_hbm.at[pl.ds(base, TPW)], idx_v.at[pl.ds(0, TPW)])

    def start_in(c, b):
        pltpu.async_copy(tokens_hbm.at[pl.ds(base + c * C, C)], tins[b],
                         in_sems[b])

    def start_out(c, b):
        pltpu.async_copy(tins[b], out_hbm.at[pl.ds(base + c * C, C)],
                         out_sems[b])

    def wait_in(b):
        pltpu.make_async_copy(tokens_hbm.at[pl.ds(base, C)], tins[b],
                              in_sems[b]).wait()

    def wait_out(b):
        pltpu.make_async_copy(tins[b], out_hbm.at[pl.ds(base, C)],
                              out_sems[b]).wait()

    def compute(c, b):
        tin = tins[b]

        def row_body(i, carry):
            row = idx_v[pl.ds(c * C + i, LANES)][0]
            for j0 in range(0, GROUPS, 32):
                gs = [table_v[row, pl.ds((j0 + j) * LANES, LANES)]
                      for j in range(32)]
                for j in range(32):
                    plsc.addupdate(tin.at[i, pl.ds((j0 + j) * LANES, LANES)],
                                   gs[j])
            return carry

        lax.fori_loop(0, C, row_body, 0)

    start_in(0, 0)
    start_in(1, 1)
    tbl_copy.wait()

    def turn(c, b, first, last):
        wait_in(b)
        bp = (b + 2) % NBUF
        if not (last and b >= 2):
            if not (first and b < 2):
                wait_out(bp)
            start_in(c + 2, bp)
        compute(c, b)
        start_out(c, b)

    def group(g, first, last):
        for b in range(NBUF):
            turn(g * NBUF + b, b, first, last)

    def mid(g, carry):
        group(g, False, False)
        return carry

    group(0, True, False)
    lax.fori_loop(1, NGROUP - 1, mid, 0)
    group(NGROUP - 1, False, True)

    for b in range(NBUF):
        wait_out(b)


@jax.jit
def _run(tokens2d, idx1d, table):
    mesh = plsc.VectorSubcoreMesh(core_axis_name="c", subcore_axis_name="s")
    f = pl.kernel(
        _body,
        out_type=jax.ShapeDtypeStruct((N, D), jnp.float32),
        mesh=mesh,
        compiler_params=pltpu.CompilerParams(needs_layout_passes=False),
        scratch_types=[
            pltpu.VMEM((TPW + LANES,), jnp.int32),
            pltpu.VMEM((MAX_ROWS, D), jnp.float32),
            pltpu.VMEM((C, D), jnp.float32),
            pltpu.VMEM((C, D), jnp.float32),
            pltpu.VMEM((C, D), jnp.float32),
            pltpu.VMEM((C, D), jnp.float32),
            pltpu.SemaphoreType.DMA,
            pltpu.SemaphoreType.DMA,
            pltpu.SemaphoreType.DMA,
            pltpu.SemaphoreType.DMA,
            pltpu.SemaphoreType.DMA,
            pltpu.SemaphoreType.DMA,
            pltpu.SemaphoreType.DMA,
            pltpu.SemaphoreType.DMA,
            pltpu.SemaphoreType.DMA,
        ],
    )
    return f(tokens2d, idx1d, table)


def kernel(tokens, example_positions, pos_table):
    tokens2d = tokens.reshape(N, D)
    idx1d = example_positions.reshape(N).astype(jnp.int32)
    out = _run(tokens2d, idx1d, pos_table)
    return out.reshape(B, S, D)
